# baseline (speedup 1.0000x reference)
import jax
import jax.numpy as jnp
from jax.experimental import pallas as pl
from jax.experimental.pallas import tpu as pltpu

_EPS = 1e-7
_IGNORE = 0.7
_N = 22743
_N_OBJ = 17328
_C = 85
_ANCHORS = ((10.0, 13.0), (16.0, 30.0), (33.0, 23.0))

_BR = 17328
_JOBJ = _N_OBJ // _BR
_NJ = -(-_N // _BR)

_DN_T = (((1,), (1,)), ((), ()))


def _loss_kernel(x_ref, t_ref, out_ref):
    b = pl.program_id(0)
    j = pl.program_id(1)

    xv = x_ref[0]
    tv = t_ref[0]

    one = jnp.float32(1.0)
    zero = jnp.float32(0.0)
    eps = jnp.float32(_EPS)
    p = jnp.clip(xv, eps, one - eps)
    log1mp = jnp.log(one - p)
    nbce = tv * (jnp.log(p) - log1mp) + log1mp

    sub4 = jax.lax.broadcasted_iota(jnp.int32, (4, _C), 0)
    lan4 = jax.lax.broadcasted_iota(jnp.int32, (4, _C), 1)
    w4 = jnp.where(
        ((sub4 == 0) & (lan4 < 2))
        | ((sub4 == 1) & ((lan4 == 2) | (lan4 == 3)))
        | ((sub4 == 2) & (lan4 == 4))
        | (sub4 == 3),
        one, zero)
    sub3 = jax.lax.broadcasted_iota(jnp.int32, (3, _C), 0)
    lan3 = jax.lax.broadcasted_iota(jnp.int32, (3, _C), 1)
    w3 = jnp.where(lan3 == sub3 + 2, one, zero)

    r4 = jax.lax.dot_general(w4, nbce, _DN_T,
                             preferred_element_type=jnp.float32)
    t3 = jax.lax.dot_general(w3, tv, _DN_T,
                             preferred_element_type=jnp.float32)

    lane = jax.lax.broadcasted_iota(jnp.int32, (1, _BR), 1)
    validb = (j * _BR + lane) < _N
    mt = jnp.where(t3[2:3, :] > zero, one, zero)
    mtv = jnp.where(validb, mt, zero)

    s_xy = -jnp.sum(jnp.where(validb, r4[0:1, :] * mt, zero))
    s_wh = -jnp.sum(jnp.where(validb, r4[1:2, :] * mt, zero))
    s_cls = -jnp.sum(jnp.where(validb, r4[2:3, :] * mt, zero))
    s_m = jnp.sum(mtv)

    acc_lane = jax.lax.broadcasted_iota(jnp.int32, (8, 128), 1)

    @pl.when((b == 0) & (j == 0))
    def _init():
        out_ref[...] = jnp.zeros_like(out_ref)

    out_ref[...] += (
        jnp.where(acc_lane == 0, s_xy, zero)
        + jnp.where(acc_lane == 1, s_wh, zero)
        + jnp.where(acc_lane == 2, s_cls, zero)
        + jnp.where(acc_lane == 3, s_m, zero)
    )

    @pl.when(j < _JOBJ)
    def _obj():
        lf = lane.astype(jnp.float32)
        kf = lf - 3.0 * jnp.floor(lf * (1.0 / 3.0) + 0.15)
        k0 = kf < 0.5
        k1 = (kf >= 0.5) & (kf < 1.5)

        aw = jnp.where(k0, _ANCHORS[0][0], jnp.where(k1, _ANCHORS[1][0], _ANCHORS[2][0]))
        ah = jnp.where(k0, _ANCHORS[0][1], jnp.where(k1, _ANCHORS[1][1], _ANCHORS[2][1]))
        area = aw * ah
        w = t3[0:1, :]
        h = t3[1:2, :]
        inter = jnp.minimum(aw, w) * jnp.minimum(ah, h)
        iou = inter / (area + w * h - inter + jnp.float32(1e-16))

        prev1 = pltpu.roll(iou, 1, 1)
        prev2 = pltpu.roll(iou, 2, 1)
        next1 = pltpu.roll(iou, _BR - 1, 1)
        next2 = pltpu.roll(iou, _BR - 2, 1)
        ciou0 = jnp.where(k0, iou, jnp.where(k1, prev1, prev2))
        ciou1 = jnp.where(k0, next1, jnp.where(k1, iou, prev1))
        ciou2 = jnp.where(k0, next2, jnp.where(k1, next1, iou))
        b0 = (ciou0 >= ciou1) & (ciou0 >= ciou2)
        b1 = jnp.logical_not(b0) & (ciou1 >= ciou2)
        b2 = jnp.logical_not(b0 | b1)
        is_best = (k0 & b0) | (k1 & b1) | ((kf >= 1.5) & b2)
        maskr = jnp.where(is_best | (iou <= jnp.float32(_IGNORE)), one, zero)

        c0 = -jnp.log(one - eps)
        s_obj = -jnp.sum(maskr * r4[3:4, :]) + jnp.float32(_C) * c0 * (
            jnp.float32(_BR) - jnp.sum(maskr))
        out_ref[...] += jnp.where(acc_lane == 4, s_obj, zero)


@jax.jit
def kernel(x, target):
    B = x.shape[0]

    out = pl.pallas_call(
        _loss_kernel,
        grid=(B, _NJ),
        in_specs=[
            pl.BlockSpec((1, _BR, _C), lambda b, j: (b, j, 0)),
            pl.BlockSpec((1, _BR, _C), lambda b, j: (b, j, 0)),
        ],
        out_specs=pl.BlockSpec((8, 128), lambda b, j: (0, 0)),
        out_shape=jax.ShapeDtypeStruct((8, 128), jnp.float32),
    )(x, target)

    s_xy = out[0, 0]
    s_wh = out[0, 1]
    s_cls = out[0, 2]
    s_m = out[0, 3]
    s_obj = out[0, 4]

    n_obj = jnp.float32(B * _N_OBJ * _C)
    return (s_xy + s_wh) / (2.0 * s_m) + s_cls / s_m + s_obj / n_obj


# speedup vs baseline: 1.0806x; 1.0806x over previous
import jax
import jax.numpy as jnp
from jax.experimental import pallas as pl
from jax.experimental.pallas import tpu as pltpu

_EPS = 1e-7
_IGNORE = 0.7
_N = 22743
_N_OBJ = 17328
_C = 85
_ANCHORS = ((10.0, 13.0), (16.0, 30.0), (33.0, 23.0))

_BR = 8664
_JOBJ = _N_OBJ // _BR
_NJ = -(-_N // _BR)

_DN_T = (((1,), (1,)), ((), ()))


def _loss_kernel(x_ref, t_ref, out_ref):
    j = pl.program_id(1)

    xv = x_ref[0]
    tv = t_ref[0]

    one = jnp.float32(1.0)
    zero = jnp.float32(0.0)
    eps = jnp.float32(_EPS)
    p = jnp.clip(xv, eps, one - eps)
    log1mp = jnp.log(one - p)
    nbce = tv * (jnp.log(p) - log1mp) + log1mp

    sub4 = jax.lax.broadcasted_iota(jnp.int32, (4, _C), 0)
    lan4 = jax.lax.broadcasted_iota(jnp.int32, (4, _C), 1)
    w4 = jnp.where(
        ((sub4 == 0) & (lan4 < 2))
        | ((sub4 == 1) & ((lan4 == 2) | (lan4 == 3)))
        | ((sub4 == 2) & (lan4 == 4))
        | (sub4 == 3),
        one, zero)
    sub3 = jax.lax.broadcasted_iota(jnp.int32, (3, _C), 0)
    lan3 = jax.lax.broadcasted_iota(jnp.int32, (3, _C), 1)
    w3 = jnp.where(lan3 == sub3 + 2, one, zero)

    r4 = jax.lax.dot_general(w4, nbce, _DN_T,
                             preferred_element_type=jnp.float32)
    t3 = jax.lax.dot_general(w3, tv, _DN_T,
                             preferred_element_type=jnp.float32)

    lane = jax.lax.broadcasted_iota(jnp.int32, (1, _BR), 1)
    validb = (j * _BR + lane) < _N
    mt = jnp.where(t3[2:3, :] > zero, one, zero)
    mtv = jnp.where(validb, mt, zero)

    s_xy = -jnp.sum(jnp.where(validb, r4[0:1, :] * mt, zero))
    s_wh = -jnp.sum(jnp.where(validb, r4[1:2, :] * mt, zero))
    s_cls = -jnp.sum(jnp.where(validb, r4[2:3, :] * mt, zero))
    s_m = jnp.sum(mtv)

    acc_lane = jax.lax.broadcasted_iota(jnp.int32, (8, 128), 1)

    @pl.when(j == 0)
    def _init():
        out_ref[0] = jnp.zeros((8, 128), jnp.float32)

    out_ref[0] += (
        jnp.where(acc_lane == 0, s_xy, zero)
        + jnp.where(acc_lane == 1, s_wh, zero)
        + jnp.where(acc_lane == 2, s_cls, zero)
        + jnp.where(acc_lane == 3, s_m, zero)
    )

    @pl.when(j < _JOBJ)
    def _obj():
        lf = lane.astype(jnp.float32)
        kf = lf - 3.0 * jnp.floor(lf * (1.0 / 3.0) + 0.15)
        k0 = kf < 0.5
        k1 = (kf >= 0.5) & (kf < 1.5)

        aw = jnp.where(k0, _ANCHORS[0][0], jnp.where(k1, _ANCHORS[1][0], _ANCHORS[2][0]))
        ah = jnp.where(k0, _ANCHORS[0][1], jnp.where(k1, _ANCHORS[1][1], _ANCHORS[2][1]))
        area = aw * ah
        w = t3[0:1, :]
        h = t3[1:2, :]
        inter = jnp.minimum(aw, w) * jnp.minimum(ah, h)
        iou = inter / (area + w * h - inter + jnp.float32(1e-16))

        prev1 = pltpu.roll(iou, 1, 1)
        prev2 = pltpu.roll(iou, 2, 1)
        next1 = pltpu.roll(iou, _BR - 1, 1)
        next2 = pltpu.roll(iou, _BR - 2, 1)
        ciou0 = jnp.where(k0, iou, jnp.where(k1, prev1, prev2))
        ciou1 = jnp.where(k0, next1, jnp.where(k1, iou, prev1))
        ciou2 = jnp.where(k0, next2, jnp.where(k1, next1, iou))
        b0 = (ciou0 >= ciou1) & (ciou0 >= ciou2)
        b1 = jnp.logical_not(b0) & (ciou1 >= ciou2)
        b2 = jnp.logical_not(b0 | b1)
        is_best = (k0 & b0) | (k1 & b1) | ((kf >= 1.5) & b2)
        maskr = jnp.where(is_best | (iou <= jnp.float32(_IGNORE)), one, zero)

        c0 = -jnp.log(one - eps)
        s_obj = -jnp.sum(maskr * r4[3:4, :]) + jnp.float32(_C) * c0 * (
            jnp.float32(_BR) - jnp.sum(maskr))
        out_ref[0] += jnp.where(acc_lane == 4, s_obj, zero)


@jax.jit
def kernel(x, target):
    B = x.shape[0]

    out = pl.pallas_call(
        _loss_kernel,
        grid=(B, _NJ),
        in_specs=[
            pl.BlockSpec((1, _BR, _C), lambda b, j: (b, j, 0)),
            pl.BlockSpec((1, _BR, _C), lambda b, j: (b, j, 0)),
        ],
        out_specs=pl.BlockSpec((1, 8, 128), lambda b, j: (b, 0, 0)),
        out_shape=jax.ShapeDtypeStruct((B, 8, 128), jnp.float32),
        compiler_params=pltpu.CompilerParams(
            dimension_semantics=("parallel", "arbitrary")),
    )(x, target)

    sums = out[:, 0, :].sum(axis=0)
    s_xy = sums[0]
    s_wh = sums[1]
    s_cls = sums[2]
    s_m = sums[3]
    s_obj = sums[4]

    n_obj = jnp.float32(B * _N_OBJ * _C)
    return (s_xy + s_wh) / (2.0 * s_m) + s_cls / s_m + s_obj / n_obj


# speedup vs baseline: 1.1482x; 1.0626x over previous
import jax
import jax.numpy as jnp
from jax.experimental import pallas as pl
from jax.experimental.pallas import tpu as pltpu

_EPS = 1e-7
_IGNORE = 0.7
_N = 22743
_N_OBJ = 17328
_N_TAIL = _N - _N_OBJ
_C = 85
_ANCHORS = ((10.0, 13.0), (16.0, 30.0), (33.0, 23.0))

_BR = 8664
_NJ = _N_OBJ // _BR

_DN_T = (((1,), (1,)), ((), ()))


def _loss_kernel(x_ref, t_ref, xc_ref, tc_ref, out_ref):
    b = pl.program_id(0)
    j = pl.program_id(1)

    xv = x_ref[0]
    tv = t_ref[0]

    one = jnp.float32(1.0)
    zero = jnp.float32(0.0)
    eps = jnp.float32(_EPS)
    p = jnp.clip(xv, eps, one - eps)
    log1mp = jnp.log(one - p)
    nbce = tv * (jnp.log(p) - log1mp) + log1mp

    sub4 = jax.lax.broadcasted_iota(jnp.int32, (4, _C), 0)
    lan4 = jax.lax.broadcasted_iota(jnp.int32, (4, _C), 1)
    w4 = jnp.where(
        ((sub4 == 0) & (lan4 < 2))
        | ((sub4 == 1) & ((lan4 == 2) | (lan4 == 3)))
        | ((sub4 == 2) & (lan4 == 4))
        | (sub4 == 3),
        one, zero)
    sub3 = jax.lax.broadcasted_iota(jnp.int32, (3, _C), 0)
    lan3 = jax.lax.broadcasted_iota(jnp.int32, (3, _C), 1)
    w3 = jnp.where(lan3 == sub3 + 2, one, zero)

    r4 = jax.lax.dot_general(w4, nbce, _DN_T,
                             preferred_element_type=jnp.float32)
    t3 = jax.lax.dot_general(w3, tv, _DN_T,
                             preferred_element_type=jnp.float32)

    mt = jnp.where(t3[2:3, :] > zero, one, zero)

    s_xy = -jnp.sum(r4[0:1, :] * mt)
    s_wh = -jnp.sum(r4[1:2, :] * mt)
    s_cls = -jnp.sum(r4[2:3, :] * mt)
    s_m = jnp.sum(mt)

    lane = jax.lax.broadcasted_iota(jnp.int32, (1, _BR), 1)
    lf = lane.astype(jnp.float32)
    kf = lf - 3.0 * jnp.floor(lf * (1.0 / 3.0) + 0.15)
    k0 = kf < 0.5
    k1 = (kf >= 0.5) & (kf < 1.5)

    aw = jnp.where(k0, _ANCHORS[0][0], jnp.where(k1, _ANCHORS[1][0], _ANCHORS[2][0]))
    ah = jnp.where(k0, _ANCHORS[0][1], jnp.where(k1, _ANCHORS[1][1], _ANCHORS[2][1]))
    area = aw * ah
    w = t3[0:1, :]
    h = t3[1:2, :]
    inter = jnp.minimum(aw, w) * jnp.minimum(ah, h)
    iou = inter / (area + w * h - inter + jnp.float32(1e-16))

    prev1 = pltpu.roll(iou, 1, 1)
    prev2 = pltpu.roll(iou, 2, 1)
    next1 = pltpu.roll(iou, _BR - 1, 1)
    next2 = pltpu.roll(iou, _BR - 2, 1)
    ciou0 = jnp.where(k0, iou, jnp.where(k1, prev1, prev2))
    ciou1 = jnp.where(k0, next1, jnp.where(k1, iou, prev1))
    ciou2 = jnp.where(k0, next2, jnp.where(k1, next1, iou))
    b0 = (ciou0 >= ciou1) & (ciou0 >= ciou2)
    b1 = jnp.logical_not(b0) & (ciou1 >= ciou2)
    b2 = jnp.logical_not(b0 | b1)
    is_best = (k0 & b0) | (k1 & b1) | ((kf >= 1.5) & b2)
    maskr = jnp.where(is_best | (iou <= jnp.float32(_IGNORE)), one, zero)

    c0 = -jnp.log(one - eps)
    s_obj = -jnp.sum(maskr * r4[3:4, :]) + jnp.float32(_C) * c0 * (
        jnp.float32(_BR) - jnp.sum(maskr))

    acc_lane = jax.lax.broadcasted_iota(jnp.int32, (8, 128), 1)
    partial = (
        jnp.where(acc_lane == 0, s_xy, zero)
        + jnp.where(acc_lane == 1, s_wh, zero)
        + jnp.where(acc_lane == 2, s_cls, zero)
        + jnp.where(acc_lane == 3, s_m, zero)
        + jnp.where(acc_lane == 4, s_obj, zero)
    )

    @pl.when((b == 0) & (j == 0))
    def _init():
        xcv = xc_ref[...]
        tcv = tc_ref[...]
        pc = jnp.clip(xcv, eps, one - eps)
        l1c = jnp.log(one - pc)
        nb5 = tcv * (jnp.log(pc) - l1c) + l1c
        m5 = jnp.where(tcv[4:5, :] > zero, one, zero)
        t_xy = -jnp.sum((nb5[0:1, :] + nb5[1:2, :]) * m5)
        t_wh = -jnp.sum((nb5[2:3, :] + nb5[3:4, :]) * m5)
        t_cls = -jnp.sum(nb5[4:5, :] * m5)
        t_m = jnp.sum(m5)
        out_ref[...] = (
            jnp.where(acc_lane == 0, t_xy, zero)
            + jnp.where(acc_lane == 1, t_wh, zero)
            + jnp.where(acc_lane == 2, t_cls, zero)
            + jnp.where(acc_lane == 3, t_m, zero)
        )

    out_ref[...] += partial


@jax.jit
def kernel(x, target):
    B = x.shape[0]
    xc = x[:, _N_OBJ:, :5].transpose(2, 0, 1).reshape(5, B * _N_TAIL)
    tc = target[:, _N_OBJ:, :5].transpose(2, 0, 1).reshape(5, B * _N_TAIL)

    nt = B * _N_TAIL

    out = pl.pallas_call(
        _loss_kernel,
        grid=(B, _NJ),
        in_specs=[
            pl.BlockSpec((1, _BR, _C), lambda b, j: (b, j, 0)),
            pl.BlockSpec((1, _BR, _C), lambda b, j: (b, j, 0)),
            pl.BlockSpec((5, nt), lambda b, j: (0, 0)),
            pl.BlockSpec((5, nt), lambda b, j: (0, 0)),
        ],
        out_specs=pl.BlockSpec((8, 128), lambda b, j: (0, 0)),
        out_shape=jax.ShapeDtypeStruct((8, 128), jnp.float32),
    )(x, target, xc, tc)

    s_xy = out[0, 0]
    s_wh = out[0, 1]
    s_cls = out[0, 2]
    s_m = out[0, 3]
    s_obj = out[0, 4]

    n_obj = jnp.float32(B * _N_OBJ * _C)
    return (s_xy + s_wh) / (2.0 * s_m) + s_cls / s_m + s_obj / n_obj


# speedup vs baseline: 1.1488x; 1.0005x over previous
import jax
import jax.numpy as jnp
from jax.experimental import pallas as pl
from jax.experimental.pallas import tpu as pltpu

_EPS = 1e-7
_IGNORE = 0.7
_N = 22743
_N_OBJ = 17328
_N_TAIL = _N - _N_OBJ
_C = 85
_ANCHORS = ((10.0, 13.0), (16.0, 30.0), (33.0, 23.0))

_BR = 8664
_NJ = _N_OBJ // _BR

_DN_T = (((1,), (1,)), ((), ()))


def _loss_kernel(x_ref, t_ref, xc_ref, tc_ref, out_ref):
    b = pl.program_id(0)
    j = pl.program_id(1)

    xv = x_ref[0]
    tv = t_ref[0]

    one = jnp.float32(1.0)
    zero = jnp.float32(0.0)
    eps = jnp.float32(_EPS)
    p = jnp.clip(xv, eps, one - eps)
    log1mp = jnp.log(one - p)
    nbce = tv * (jnp.log(p) - log1mp) + log1mp

    sub4 = jax.lax.broadcasted_iota(jnp.int32, (4, _C), 0)
    lan4 = jax.lax.broadcasted_iota(jnp.int32, (4, _C), 1)
    w4 = jnp.where(
        ((sub4 == 0) & (lan4 < 2))
        | ((sub4 == 1) & ((lan4 == 2) | (lan4 == 3)))
        | ((sub4 == 2) & (lan4 == 4))
        | (sub4 == 3),
        one, zero)
    sub3 = jax.lax.broadcasted_iota(jnp.int32, (3, _C), 0)
    lan3 = jax.lax.broadcasted_iota(jnp.int32, (3, _C), 1)
    w3 = jnp.where(lan3 == sub3 + 2, one, zero)

    r4 = jax.lax.dot_general(w4, nbce, _DN_T,
                             precision=jax.lax.Precision.DEFAULT,
                             preferred_element_type=jnp.float32)
    t3 = jax.lax.dot_general(w3, tv, _DN_T,
                             precision=jax.lax.Precision.DEFAULT,
                             preferred_element_type=jnp.float32)

    mt = jnp.where(t3[2:3, :] > zero, one, zero)

    s_xy = -jnp.sum(r4[0:1, :] * mt)
    s_wh = -jnp.sum(r4[1:2, :] * mt)
    s_cls = -jnp.sum(r4[2:3, :] * mt)
    s_m = jnp.sum(mt)

    lane = jax.lax.broadcasted_iota(jnp.int32, (1, _BR), 1)
    lf = lane.astype(jnp.float32)
    kf = lf - 3.0 * jnp.floor(lf * (1.0 / 3.0) + 0.15)
    k0 = kf < 0.5
    k1 = (kf >= 0.5) & (kf < 1.5)

    aw = jnp.where(k0, _ANCHORS[0][0], jnp.where(k1, _ANCHORS[1][0], _ANCHORS[2][0]))
    ah = jnp.where(k0, _ANCHORS[0][1], jnp.where(k1, _ANCHORS[1][1], _ANCHORS[2][1]))
    area = aw * ah
    w = t3[0:1, :]
    h = t3[1:2, :]
    inter = jnp.minimum(aw, w) * jnp.minimum(ah, h)
    iou = inter / (area + w * h - inter + jnp.float32(1e-16))

    prev1 = pltpu.roll(iou, 1, 1)
    prev2 = pltpu.roll(iou, 2, 1)
    next1 = pltpu.roll(iou, _BR - 1, 1)
    next2 = pltpu.roll(iou, _BR - 2, 1)
    ciou0 = jnp.where(k0, iou, jnp.where(k1, prev1, prev2))
    ciou1 = jnp.where(k0, next1, jnp.where(k1, iou, prev1))
    ciou2 = jnp.where(k0, next2, jnp.where(k1, next1, iou))
    b0 = (ciou0 >= ciou1) & (ciou0 >= ciou2)
    b1 = jnp.logical_not(b0) & (ciou1 >= ciou2)
    b2 = jnp.logical_not(b0 | b1)
    is_best = (k0 & b0) | (k1 & b1) | ((kf >= 1.5) & b2)
    maskr = jnp.where(is_best | (iou <= jnp.float32(_IGNORE)), one, zero)

    c0 = -jnp.log(one - eps)
    s_obj = -jnp.sum(maskr * r4[3:4, :]) + jnp.float32(_C) * c0 * (
        jnp.float32(_BR) - jnp.sum(maskr))

    acc_lane = jax.lax.broadcasted_iota(jnp.int32, (8, 128), 1)
    partial = (
        jnp.where(acc_lane == 0, s_xy, zero)
        + jnp.where(acc_lane == 1, s_wh, zero)
        + jnp.where(acc_lane == 2, s_cls, zero)
        + jnp.where(acc_lane == 3, s_m, zero)
        + jnp.where(acc_lane == 4, s_obj, zero)
    )

    @pl.when((b == 0) & (j == 0))
    def _init():
        xcv = xc_ref[...]
        tcv = tc_ref[...]
        pc = jnp.clip(xcv, eps, one - eps)
        l1c = jnp.log(one - pc)
        nb5 = tcv * (jnp.log(pc) - l1c) + l1c
        m5 = jnp.where(tcv[4:5, :] > zero, one, zero)
        t_xy = -jnp.sum((nb5[0:1, :] + nb5[1:2, :]) * m5)
        t_wh = -jnp.sum((nb5[2:3, :] + nb5[3:4, :]) * m5)
        t_cls = -jnp.sum(nb5[4:5, :] * m5)
        t_m = jnp.sum(m5)
        out_ref[...] = (
            jnp.where(acc_lane == 0, t_xy, zero)
            + jnp.where(acc_lane == 1, t_wh, zero)
            + jnp.where(acc_lane == 2, t_cls, zero)
            + jnp.where(acc_lane == 3, t_m, zero)
        )

    out_ref[...] += partial


@jax.jit
def kernel(x, target):
    B = x.shape[0]
    xc = x[:, _N_OBJ:, :5].transpose(2, 0, 1).reshape(5, B * _N_TAIL)
    tc = target[:, _N_OBJ:, :5].transpose(2, 0, 1).reshape(5, B * _N_TAIL)

    nt = B * _N_TAIL

    out = pl.pallas_call(
        _loss_kernel,
        grid=(B, _NJ),
        in_specs=[
            pl.BlockSpec((1, _BR, _C), lambda b, j: (b, j, 0)),
            pl.BlockSpec((1, _BR, _C), lambda b, j: (b, j, 0)),
            pl.BlockSpec((5, nt), lambda b, j: (0, 0)),
            pl.BlockSpec((5, nt), lambda b, j: (0, 0)),
        ],
        out_specs=pl.BlockSpec((8, 128), lambda b, j: (0, 0)),
        out_shape=jax.ShapeDtypeStruct((8, 128), jnp.float32),
    )(x, target, xc, tc)

    s_xy = out[0, 0]
    s_wh = out[0, 1]
    s_cls = out[0, 2]
    s_m = out[0, 3]
    s_obj = out[0, 4]

    n_obj = jnp.float32(B * _N_OBJ * _C)
    return (s_xy + s_wh) / (2.0 * s_m) + s_cls / s_m + s_obj / n_obj
